# baseline (speedup 1.0000x reference)
import functools

import jax
import jax.numpy as jnp
from jax.experimental import pallas as pl
from jax.experimental.pallas import tpu as pltpu

N = 8192
NFEAT = 256
NHID = 64
ALPHA = 0.2


def _elu(x):
    return jnp.where(x > 0, x, jnp.exp(x) - 1.0)


def _proj_kernel(x_ref, w_ref, b_ref, amat_ref, h_ref, sc_ref, mx_ref):
    i = pl.program_id(0)
    h = jnp.dot(x_ref[...], w_ref[...], preferred_element_type=jnp.float32)
    h = h + b_ref[...]
    h_ref[...] = h.astype(jnp.bfloat16)
    sc = jnp.dot(h, amat_ref[...], preferred_element_type=jnp.float32)
    sc_ref[...] = sc
    cm = jnp.max(sc, axis=0, keepdims=True)

    @pl.when(i == 0)
    def _():
        mx_ref[...] = cm

    @pl.when(i > 0)
    def _():
        mx_ref[...] = jnp.maximum(mx_ref[...], cm)


def _project(x, w_aug, bias, amat, block_rows=1024):
    n, k = x.shape
    f = w_aug.shape[1]
    return pl.pallas_call(
        _proj_kernel,
        grid=(n // block_rows,),
        in_specs=[
            pl.BlockSpec((block_rows, k), lambda i: (i, 0)),
            pl.BlockSpec((k, f), lambda i: (0, 0)),
            pl.BlockSpec((1, f), lambda i: (0, 0)),
            pl.BlockSpec((f, 128), lambda i: (0, 0)),
        ],
        out_specs=[
            pl.BlockSpec((block_rows, f), lambda i: (i, 0)),
            pl.BlockSpec((block_rows, 128), lambda i: (i, 0)),
            pl.BlockSpec((1, 128), lambda i: (0, 0)),
        ],
        out_shape=[
            jax.ShapeDtypeStruct((n, f), jnp.bfloat16),
            jax.ShapeDtypeStruct((n, 128), jnp.float32),
            jax.ShapeDtypeStruct((1, 128), jnp.float32),
        ],
        compiler_params=pltpu.CompilerParams(
            dimension_semantics=("arbitrary",),
        ),
    )(x, w_aug, bias, amat)


def _softmax_weights(adj, sc_ref, mx_ref, ert_ref, h):
    erm = mx_ref[0:1, 2 * h + 1:2 * h + 2]
    t = sc_ref[:, 2 * h:2 * h + 1] + erm
    g = jnp.minimum(jnp.exp(-0.8 * t), 240.0).astype(jnp.bfloat16)
    d0 = ert_ref[h:h + 1, :] - erm
    b = jnp.exp(d0).astype(jnp.bfloat16)
    d = jnp.exp(ALPHA * d0).astype(jnp.bfloat16)
    return adj * jnp.maximum(b, g * d)


def _head_out(p, wh_ref, h):
    acc = jnp.dot(p, wh_ref[:, 128 * h:128 * (h + 1)],
                  preferred_element_type=jnp.float32)
    s = jnp.maximum(acc[:, NHID:NHID + 1], 1e-30)
    return acc[:, :NHID] * (1.0 / s)


def _attn1_kernel(adj_ref, wh_ref, sc_ref, mx_ref, ert_ref,
                  w2_ref, b2_ref, am2_ref,
                  adj8_ref, wh2_ref, sc2_ref, mx2_ref):
    i = pl.program_id(0)
    adj = adj_ref[...].astype(jnp.bfloat16)
    adj8_ref[...] = adj.astype(jnp.int8)
    outs = []
    for h in range(2):
        p = _softmax_weights(adj, sc_ref, mx_ref, ert_ref, h)
        outs.append(_elu(_head_out(p, wh_ref, h)))
    h2 = (jnp.dot(outs[0], w2_ref[0:NHID, :],
                  preferred_element_type=jnp.float32)
          + jnp.dot(outs[1], w2_ref[NHID:2 * NHID, :],
                    preferred_element_type=jnp.float32)
          + b2_ref[...])
    wh2_ref[...] = h2.astype(jnp.bfloat16)
    sc2 = jnp.dot(h2, am2_ref[...], preferred_element_type=jnp.float32)
    sc2_ref[...] = sc2
    cm = jnp.max(sc2, axis=0, keepdims=True)

    @pl.when(i == 0)
    def _():
        mx2_ref[...] = cm

    @pl.when(i > 0)
    def _():
        mx2_ref[...] = jnp.maximum(mx2_ref[...], cm)


def _attn1(adj, wh_aug, scores, colmax, er_t, w2, b2, am2, br=512):
    n = adj.shape[0]
    return pl.pallas_call(
        _attn1_kernel,
        grid=(n // br,),
        in_specs=[
            pl.BlockSpec((br, n), lambda i: (i, 0)),
            pl.BlockSpec((n, 256), lambda i: (0, 0)),
            pl.BlockSpec((br, 128), lambda i: (i, 0)),
            pl.BlockSpec((1, 128), lambda i: (0, 0)),
            pl.BlockSpec((8, n), lambda i: (0, 0)),
            pl.BlockSpec((128, 128), lambda i: (0, 0)),
            pl.BlockSpec((1, 128), lambda i: (0, 0)),
            pl.BlockSpec((128, 128), lambda i: (0, 0)),
        ],
        out_specs=[
            pl.BlockSpec((br, n), lambda i: (i, 0)),
            pl.BlockSpec((br, 128), lambda i: (i, 0)),
            pl.BlockSpec((br, 128), lambda i: (i, 0)),
            pl.BlockSpec((1, 128), lambda i: (0, 0)),
        ],
        out_shape=[
            jax.ShapeDtypeStruct((n, n), jnp.int8),
            jax.ShapeDtypeStruct((n, 128), jnp.bfloat16),
            jax.ShapeDtypeStruct((n, 128), jnp.float32),
            jax.ShapeDtypeStruct((1, 128), jnp.float32),
        ],
        compiler_params=pltpu.CompilerParams(
            dimension_semantics=("arbitrary",),
        ),
    )(adj, wh_aug, scores, colmax, er_t, w2, b2, am2)


def _attn2_kernel(adj_ref, wh_ref, sc_ref, mx_ref, ert_ref, out_ref):
    adj = adj_ref[...].astype(jnp.bfloat16)
    p = _softmax_weights(adj, sc_ref, mx_ref, ert_ref, 0)
    o = _elu(_head_out(p, wh_ref, 0))
    mx = jnp.max(o, axis=1, keepdims=True)
    lse = jnp.log(jnp.sum(jnp.exp(o - mx), axis=1, keepdims=True))
    out_ref[...] = o - mx - lse


def _attn2(adj8, wh_aug, scores, colmax, er_t, br=1024):
    n = adj8.shape[0]
    return pl.pallas_call(
        _attn2_kernel,
        grid=(n // br,),
        in_specs=[
            pl.BlockSpec((br, n), lambda i: (i, 0)),
            pl.BlockSpec((n, 128), lambda i: (0, 0)),
            pl.BlockSpec((br, 128), lambda i: (i, 0)),
            pl.BlockSpec((1, 128), lambda i: (0, 0)),
            pl.BlockSpec((8, n), lambda i: (0, 0)),
        ],
        out_specs=pl.BlockSpec((br, NHID), lambda i: (i, 0)),
        out_shape=jax.ShapeDtypeStruct((n, NHID), jnp.float32),
        compiler_params=pltpu.CompilerParams(
            dimension_semantics=("parallel",),
        ),
    )(adj8, wh_aug, scores, colmax, er_t)


def kernel(x, adj, W0_0, a0_0, W0_1, a0_1, W_out, a_out):
    f32 = jnp.float32

    w1 = jnp.zeros((NFEAT, 256), f32)
    w1 = w1.at[:, 0:NHID].set(W0_0).at[:, 128:128 + NHID].set(W0_1)
    b1 = jnp.zeros((1, 256), f32).at[0, NHID].set(1.0).at[0, 128 + NHID].set(1.0)
    amat1 = jnp.zeros((256, 128), f32)
    amat1 = (amat1.at[0:NHID, 0].set(a0_0[:NHID, 0])
                  .at[0:NHID, 1].set(a0_0[NHID:, 0])
                  .at[128:128 + NHID, 2].set(a0_1[:NHID, 0])
                  .at[128:128 + NHID, 3].set(a0_1[NHID:, 0]))
    wh1, sc1, mx1 = _project(x, w1, b1, amat1)
    er1_t = jnp.zeros((8, N), f32).at[0].set(sc1[:, 1]).at[1].set(sc1[:, 3])

    w2 = jnp.zeros((2 * NHID, 128), f32).at[:, 0:NHID].set(W_out)
    b2 = jnp.zeros((1, 128), f32).at[0, NHID].set(1.0)
    amat2 = jnp.zeros((128, 128), f32)
    amat2 = (amat2.at[0:NHID, 0].set(a_out[:NHID, 0])
                  .at[0:NHID, 1].set(a_out[NHID:, 0]))
    adj8, wh2, sc2, mx2 = _attn1(adj, wh1, sc1, mx1, er1_t, w2, b2, amat2)

    er2_t = jnp.zeros((8, N), f32).at[0].set(sc2[:, 1])
    return _attn2(adj8, wh2, sc2, mx2, er2_t)


# speedup vs baseline: 1.0068x; 1.0068x over previous
import jax
import jax.numpy as jnp
from jax.experimental import pallas as pl
from jax.experimental.pallas import tpu as pltpu

N = 8192
NFEAT = 256
NHID = 64
ALPHA = 0.2


def _elu(x):
    return jnp.where(x > 0, x, jnp.exp(x) - 1.0)


def _proj_kernel(x_ref, w_ref, b_ref, amat_ref, h_ref, sc_ref, mx_ref):
    i = pl.program_id(0)
    h = jnp.dot(x_ref[...], w_ref[...], preferred_element_type=jnp.float32)
    h = h + b_ref[...]
    h_ref[...] = h.astype(jnp.bfloat16)
    sc = jnp.dot(h, amat_ref[...], preferred_element_type=jnp.float32)
    sc_ref[...] = sc
    cm = jnp.max(sc, axis=0, keepdims=True)

    @pl.when(i == 0)
    def _():
        mx_ref[...] = cm

    @pl.when(i > 0)
    def _():
        mx_ref[...] = jnp.maximum(mx_ref[...], cm)


def _project(x, w_aug, bias, amat, block_rows=1024):
    n, k = x.shape
    f = w_aug.shape[1]
    return pl.pallas_call(
        _proj_kernel,
        grid=(n // block_rows,),
        in_specs=[
            pl.BlockSpec((block_rows, k), lambda i: (i, 0)),
            pl.BlockSpec((k, f), lambda i: (0, 0)),
            pl.BlockSpec((1, f), lambda i: (0, 0)),
            pl.BlockSpec((f, 128), lambda i: (0, 0)),
        ],
        out_specs=[
            pl.BlockSpec((block_rows, f), lambda i: (i, 0)),
            pl.BlockSpec((block_rows, 128), lambda i: (i, 0)),
            pl.BlockSpec((1, 128), lambda i: (0, 0)),
        ],
        out_shape=[
            jax.ShapeDtypeStruct((n, f), jnp.bfloat16),
            jax.ShapeDtypeStruct((n, 128), jnp.float32),
            jax.ShapeDtypeStruct((1, 128), jnp.float32),
        ],
        compiler_params=pltpu.CompilerParams(
            dimension_semantics=("arbitrary",),
        ),
    )(x, w_aug, bias, amat)


def _softmax_weights(adj, sc_ref, mx_ref, ert_ref, h):
    erm = mx_ref[0:1, 2 * h + 1:2 * h + 2]
    t = sc_ref[:, 2 * h:2 * h + 1] + erm
    g = jnp.minimum(jnp.exp(-0.8 * t), 240.0).astype(jnp.bfloat16)
    d0 = ert_ref[h:h + 1, :] - erm
    b = jnp.exp(d0).astype(jnp.bfloat16)
    d = jnp.exp(ALPHA * d0).astype(jnp.bfloat16)
    return adj * jnp.maximum(b, g * d)


def _head_out(p, wh_ref, h):
    acc = jnp.dot(p, wh_ref[:, 128 * h:128 * (h + 1)],
                  preferred_element_type=jnp.float32)
    s = jnp.maximum(acc[:, NHID:NHID + 1], 1e-30)
    return acc[:, :NHID] * (1.0 / s)


def _attn1_kernel(adj_ref, wh_ref, sc_ref, mx_ref, ert_ref,
                  w2_ref, b2_ref, am2_ref,
                  adj8_ref, wh2_ref, sc2_ref, mx2_ref):
    i = pl.program_id(0)
    adj = adj_ref[...].astype(jnp.bfloat16)
    adj8_ref[...] = adj.astype(jnp.int8)
    outs = []
    for h in range(2):
        p = _softmax_weights(adj, sc_ref, mx_ref, ert_ref, h)
        outs.append(_elu(_head_out(p, wh_ref, h)))
    h2 = (jnp.dot(outs[0], w2_ref[0:NHID, :],
                  preferred_element_type=jnp.float32)
          + jnp.dot(outs[1], w2_ref[NHID:2 * NHID, :],
                    preferred_element_type=jnp.float32)
          + b2_ref[...])
    wh2_ref[...] = h2.astype(jnp.bfloat16)
    sc2 = jnp.dot(h2, am2_ref[...], preferred_element_type=jnp.float32)
    sc2_ref[...] = sc2
    cm = jnp.max(sc2, axis=0, keepdims=True)

    @pl.when(i == 0)
    def _():
        mx2_ref[...] = cm

    @pl.when(i > 0)
    def _():
        mx2_ref[...] = jnp.maximum(mx2_ref[...], cm)


def _attn1(adj, wh_aug, scores, colmax, er_t, w2, b2, am2, br=512):
    n = adj.shape[0]
    return pl.pallas_call(
        _attn1_kernel,
        grid=(n // br,),
        in_specs=[
            pl.BlockSpec((br, n), lambda i: (i, 0)),
            pl.BlockSpec((n, 256), lambda i: (0, 0)),
            pl.BlockSpec((br, 128), lambda i: (i, 0)),
            pl.BlockSpec((1, 128), lambda i: (0, 0)),
            pl.BlockSpec((8, n), lambda i: (0, 0)),
            pl.BlockSpec((128, 128), lambda i: (0, 0)),
            pl.BlockSpec((1, 128), lambda i: (0, 0)),
            pl.BlockSpec((128, 128), lambda i: (0, 0)),
        ],
        out_specs=[
            pl.BlockSpec((br, n), lambda i: (i, 0)),
            pl.BlockSpec((br, 128), lambda i: (i, 0)),
            pl.BlockSpec((br, 128), lambda i: (i, 0)),
            pl.BlockSpec((1, 128), lambda i: (0, 0)),
        ],
        out_shape=[
            jax.ShapeDtypeStruct((n, n), jnp.int8),
            jax.ShapeDtypeStruct((n, 128), jnp.bfloat16),
            jax.ShapeDtypeStruct((n, 128), jnp.float32),
            jax.ShapeDtypeStruct((1, 128), jnp.float32),
        ],
        compiler_params=pltpu.CompilerParams(
            dimension_semantics=("arbitrary",),
        ),
    )(adj, wh_aug, scores, colmax, er_t, w2, b2, am2)


def _attn2_kernel(adj_ref, wh_ref, sc_ref, mx_ref, ert_ref, out_ref):
    adj = adj_ref[...].astype(jnp.bfloat16)
    p = _softmax_weights(adj, sc_ref, mx_ref, ert_ref, 0)
    o = _elu(_head_out(p, wh_ref, 0))
    mx = jnp.max(o, axis=1, keepdims=True)
    lse = jnp.log(jnp.sum(jnp.exp(o - mx), axis=1, keepdims=True))
    out_ref[...] = o - mx - lse


def _attn2(adj8, wh_aug, scores, colmax, er_t, br=512):
    n = adj8.shape[0]
    return pl.pallas_call(
        _attn2_kernel,
        grid=(n // br,),
        in_specs=[
            pl.BlockSpec((br, n), lambda i: (i, 0)),
            pl.BlockSpec((n, 128), lambda i: (0, 0)),
            pl.BlockSpec((br, 128), lambda i: (i, 0)),
            pl.BlockSpec((1, 128), lambda i: (0, 0)),
            pl.BlockSpec((8, n), lambda i: (0, 0)),
        ],
        out_specs=pl.BlockSpec((br, NHID), lambda i: (i, 0)),
        out_shape=jax.ShapeDtypeStruct((n, NHID), jnp.float32),
        compiler_params=pltpu.CompilerParams(
            dimension_semantics=("parallel",),
        ),
    )(adj8, wh_aug, scores, colmax, er_t)


def kernel(x, adj, W0_0, a0_0, W0_1, a0_1, W_out, a_out):
    f32 = jnp.float32

    w1 = jnp.zeros((NFEAT, 256), f32)
    w1 = w1.at[:, 0:NHID].set(W0_0).at[:, 128:128 + NHID].set(W0_1)
    b1 = jnp.zeros((1, 256), f32).at[0, NHID].set(1.0).at[0, 128 + NHID].set(1.0)
    amat1 = jnp.zeros((256, 128), f32)
    amat1 = (amat1.at[0:NHID, 0].set(a0_0[:NHID, 0])
                  .at[0:NHID, 1].set(a0_0[NHID:, 0])
                  .at[128:128 + NHID, 2].set(a0_1[:NHID, 0])
                  .at[128:128 + NHID, 3].set(a0_1[NHID:, 0]))
    wh1, sc1, mx1 = _project(x, w1, b1, amat1)
    er1_t = jnp.zeros((8, N), f32).at[0].set(sc1[:, 1]).at[1].set(sc1[:, 3])

    w2 = jnp.zeros((2 * NHID, 128), f32).at[:, 0:NHID].set(W_out)
    b2 = jnp.zeros((1, 128), f32).at[0, NHID].set(1.0)
    amat2 = jnp.zeros((128, 128), f32)
    amat2 = (amat2.at[0:NHID, 0].set(a_out[:NHID, 0])
                  .at[0:NHID, 1].set(a_out[NHID:, 0]))
    adj8, wh2, sc2, mx2 = _attn1(adj, wh1, sc1, mx1, er1_t, w2, b2, amat2)

    er2_t = jnp.zeros((8, N), f32).at[0].set(sc2[:, 1])
    return _attn2(adj8, wh2, sc2, mx2, er2_t)
